# baseline (speedup 1.0000x reference)
import functools

import jax
import jax.numpy as jnp
from jax import lax
from jax.experimental import pallas as pl
from jax.experimental.pallas import tpu as pltpu
from jax.experimental.pallas import tpu_sc as plsc

N = 10000
E = 320000
D = 128
DH = D // 2
BN_EPS = 1e-5

NC = 2
NS = 16
EPT = E // NS
CHUNK = 128
NCHUNK = (EPT + CHUNK - 1) // CHUNK
EPTP = NCHUNK * CHUNK
TRASH = N
NPAD = 10240
RPT = NPAD // NS
XPT = N // NS
PDEPTH = 3
NBUF = 6


def _sc_agg_kernel(xa_hbm, xb_hbm, src_hbm, dst_hbm, out_hbm,
                   sidx, didx, rows, agg_sh, gsem, ssem):
    c = lax.axis_index("c")
    s = lax.axis_index("s")

    @pl.when(c == 0)
    def _():
        pltpu.sync_copy(xa_hbm.at[pl.ds(s * XPT, XPT)],
                        agg_sh.at[pl.ds(s * XPT, XPT)])

    @pl.when(c == 1)
    def _():
        pltpu.sync_copy(xb_hbm.at[pl.ds(s * XPT, XPT)],
                        agg_sh.at[pl.ds(s * XPT, XPT)])

    pltpu.sync_copy(src_hbm.at[s], sidx)
    pltpu.sync_copy(dst_hbm.at[s], didx)
    plsc.subcore_barrier()

    def make_loop(x_hbm):
        def body(i, _):
            @pl.when(i < NCHUNK)
            def _():
                k = lax.rem(i, NBUF)

                @pl.when(i >= NBUF)
                def _():
                    pltpu.make_async_copy(
                        rows.at[k], agg_sh.at[didx.at[0]], ssem.at[k]
                    ).wait()

                pltpu.make_async_copy(
                    x_hbm.at[sidx.at[i]], rows.at[k], gsem.at[k]
                ).start()

            j = i - PDEPTH

            @pl.when(j >= 0)
            def _():
                kj = lax.rem(j, NBUF)
                pltpu.make_async_copy(
                    x_hbm.at[sidx.at[j]], rows.at[kj], gsem.at[kj]
                ).wait()
                pltpu.make_async_copy(
                    rows.at[kj], agg_sh.at[didx.at[j]], ssem.at[kj]
                ).start(add=True)

            return 0

        lax.fori_loop(0, NCHUNK + PDEPTH, body, 0)

        def drain(k, _):
            pltpu.make_async_copy(
                rows.at[k], agg_sh.at[didx.at[0]], ssem.at[k]
            ).wait()
            return 0

        lax.fori_loop(0, NBUF, drain, 0)

    @pl.when(c == 0)
    def _():
        make_loop(xa_hbm)

    @pl.when(c == 1)
    def _():
        make_loop(xb_hbm)

    plsc.subcore_barrier()
    pl.delay(20000)
    plsc.subcore_barrier()

    pltpu.sync_copy(agg_sh.at[pl.ds(s * RPT, RPT)],
                    out_hbm.at[pl.ds(s * RPT, RPT), pl.ds(c * DH, DH)])


_sc_agg = functools.partial(
    pl.kernel,
    out_type=jax.ShapeDtypeStruct((NPAD, D), jnp.float32),
    mesh=plsc.VectorSubcoreMesh(core_axis_name="c", subcore_axis_name="s"),
    compiler_params=pltpu.CompilerParams(use_tc_tiling_on_sc=False),
    scratch_types=[
        pltpu.VMEM((NCHUNK, CHUNK), jnp.int32),
        pltpu.VMEM((NCHUNK, CHUNK), jnp.int32),
        pltpu.VMEM((NBUF, CHUNK, DH), jnp.float32),
        pltpu.VMEM_SHARED((NPAD, DH), jnp.float32),
        pltpu.SemaphoreType.DMA((NBUF,)),
        pltpu.SemaphoreType.DMA((NBUF,)),
    ],
)(_sc_agg_kernel)


def _mlp_kernel(h0_ref, w1t_ref, b1_ref, g1_ref, be1_ref,
                w2t_ref, b2_ref, g2_ref, be2_ref, g3_ref, be3_ref, out_ref):
    invn = 1.0 / N

    def fused_layer(h, wt, b, g, be):
        m = jnp.sum(h, axis=0, keepdims=True) * invn
        gram = lax.dot_general(h, h, (((0,), (0,)), ((), ())),
                               preferred_element_type=jnp.float32) * invn
        cov = gram - m.T @ m
        var = jnp.sum((cov @ wt) * wt, axis=0, keepdims=True)
        mu = m @ wt + b
        sc = lax.rsqrt(var + BN_EPS) * g
        return jnp.maximum(
            jnp.dot(h, wt * sc, preferred_element_type=jnp.float32)
            + (b - mu) * sc + be, 0.0)

    h1 = fused_layer(h0_ref[...], w1t_ref[...], b1_ref[...],
                     g1_ref[...], be1_ref[...])
    h2 = fused_layer(h1, w2t_ref[...], b2_ref[...],
                     g2_ref[...], be2_ref[...])
    m2 = jnp.sum(h2, axis=0, keepdims=True) * invn
    ss = jnp.sum(h2 * h2, axis=0, keepdims=True) * invn
    v2 = ss - m2 * m2
    out_ref[...] = (h2 - m2) * lax.rsqrt(v2 + BN_EPS) * g3_ref[...] \
        + be3_ref[...]


def _pad_idx(a, fill):
    pad = jnp.full((NS, EPTP - EPT), fill, jnp.int32)
    return jnp.concatenate([a.reshape(NS, EPT), pad], axis=1) \
        .reshape(NS, NCHUNK, CHUNK)


def kernel(x, edge_index, W1, b1, g1, be1, W2, b2, g2, be2, g3, be3):
    src = _pad_idx(edge_index[0], 0)
    dst = _pad_idx(edge_index[1], TRASH)
    xa = x[:, :DH]
    xb = x[:, DH:]
    h0 = _sc_agg(xa, xb, src, dst)[:N]
    row = lambda v: v.reshape(1, -1)
    return pl.pallas_call(
        _mlp_kernel,
        out_shape=jax.ShapeDtypeStruct((N, D), jnp.float32),
    )(h0, W1.T, row(b1), row(g1), row(be1),
      W2.T, row(b2), row(g2), row(be2), row(g3), row(be3))


# speedup vs baseline: 1.0250x; 1.0250x over previous
import functools

import jax
import jax.numpy as jnp
from jax import lax
from jax.experimental import pallas as pl
from jax.experimental.pallas import tpu as pltpu
from jax.experimental.pallas import tpu_sc as plsc

N = 10000
E = 320000
D = 128
DH = D // 2
BN_EPS = 1e-5

NC = 2
NS = 16
EPT = E // NS
CHUNK = 80
NCHUNK = (EPT + CHUNK - 1) // CHUNK
EPTP = NCHUNK * CHUNK
TRASH = N
NPAD = 10240
RPT = NPAD // NS
XPT = N // NS
PDEPTH = 5
NBUF = 9


def _sc_agg_kernel(xa_hbm, xb_hbm, src_hbm, dst_hbm, out_hbm,
                   sidx, didx, rows, agg_sh, gsem, ssem):
    c = lax.axis_index("c")
    s = lax.axis_index("s")

    @pl.when(c == 0)
    def _():
        pltpu.sync_copy(xa_hbm.at[pl.ds(s * XPT, XPT)],
                        agg_sh.at[pl.ds(s * XPT, XPT)])

    @pl.when(c == 1)
    def _():
        pltpu.sync_copy(xb_hbm.at[pl.ds(s * XPT, XPT)],
                        agg_sh.at[pl.ds(s * XPT, XPT)])

    pltpu.sync_copy(src_hbm.at[s], sidx)
    pltpu.sync_copy(dst_hbm.at[s], didx)
    plsc.subcore_barrier()

    def make_loop(x_hbm):
        def body(i, _):
            @pl.when(i < NCHUNK)
            def _():
                k = lax.rem(i, NBUF)

                @pl.when(i >= NBUF)
                def _():
                    pltpu.make_async_copy(
                        rows.at[k], agg_sh.at[didx.at[0]], ssem.at[k]
                    ).wait()

                pltpu.make_async_copy(
                    x_hbm.at[sidx.at[i]], rows.at[k], gsem.at[k]
                ).start()

            j = i - PDEPTH

            @pl.when(j >= 0)
            def _():
                kj = lax.rem(j, NBUF)
                pltpu.make_async_copy(
                    x_hbm.at[sidx.at[j]], rows.at[kj], gsem.at[kj]
                ).wait()
                pltpu.make_async_copy(
                    rows.at[kj], agg_sh.at[didx.at[j]], ssem.at[kj]
                ).start(add=True)

            return 0

        lax.fori_loop(0, NCHUNK + PDEPTH, body, 0)

        def drain(k, _):
            pltpu.make_async_copy(
                rows.at[k], agg_sh.at[didx.at[0]], ssem.at[k]
            ).wait()
            return 0

        lax.fori_loop(0, NBUF, drain, 0)

    @pl.when(c == 0)
    def _():
        make_loop(xa_hbm)

    @pl.when(c == 1)
    def _():
        make_loop(xb_hbm)

    plsc.subcore_barrier()
    pl.delay(20000)
    plsc.subcore_barrier()

    pltpu.sync_copy(agg_sh.at[pl.ds(s * RPT, RPT)],
                    out_hbm.at[pl.ds(s * RPT, RPT), pl.ds(c * DH, DH)])


_sc_agg = functools.partial(
    pl.kernel,
    out_type=jax.ShapeDtypeStruct((NPAD, D), jnp.float32),
    mesh=plsc.VectorSubcoreMesh(core_axis_name="c", subcore_axis_name="s"),
    compiler_params=pltpu.CompilerParams(use_tc_tiling_on_sc=False),
    scratch_types=[
        pltpu.VMEM((NCHUNK, CHUNK), jnp.int32),
        pltpu.VMEM((NCHUNK, CHUNK), jnp.int32),
        pltpu.VMEM((NBUF, CHUNK, DH), jnp.float32),
        pltpu.VMEM_SHARED((NPAD, DH), jnp.float32),
        pltpu.SemaphoreType.DMA((NBUF,)),
        pltpu.SemaphoreType.DMA((NBUF,)),
    ],
)(_sc_agg_kernel)


def _mlp_kernel(h0_ref, w1t_ref, b1_ref, g1_ref, be1_ref,
                w2t_ref, b2_ref, g2_ref, be2_ref, g3_ref, be3_ref, out_ref):
    invn = 1.0 / N

    def fused_layer(h, wt, b, g, be):
        m = jnp.sum(h, axis=0, keepdims=True) * invn
        gram = lax.dot_general(h, h, (((0,), (0,)), ((), ())),
                               preferred_element_type=jnp.float32,
                               precision=lax.Precision.HIGHEST) * invn
        cov = gram - m.T @ m
        cw = jnp.dot(cov, wt, precision=lax.Precision.HIGHEST)
        var = jnp.sum(cw * wt, axis=0, keepdims=True)
        mu = m @ wt + b
        sc = lax.rsqrt(var + BN_EPS) * g
        return jnp.maximum(
            jnp.dot(h, wt * sc, preferred_element_type=jnp.float32,
                    precision=lax.Precision.HIGHEST)
            + (b - mu) * sc + be, 0.0)

    h1 = fused_layer(h0_ref[...], w1t_ref[...], b1_ref[...],
                     g1_ref[...], be1_ref[...])
    h2 = fused_layer(h1, w2t_ref[...], b2_ref[...],
                     g2_ref[...], be2_ref[...])
    m2 = jnp.sum(h2, axis=0, keepdims=True) * invn
    ss = jnp.sum(h2 * h2, axis=0, keepdims=True) * invn
    v2 = ss - m2 * m2
    out_ref[...] = (h2 - m2) * lax.rsqrt(v2 + BN_EPS) * g3_ref[...] \
        + be3_ref[...]


def _pad_idx(a, fill):
    pad = jnp.full((NS, EPTP - EPT), fill, jnp.int32)
    return jnp.concatenate([a.reshape(NS, EPT), pad], axis=1) \
        .reshape(NS, NCHUNK, CHUNK)


def kernel(x, edge_index, W1, b1, g1, be1, W2, b2, g2, be2, g3, be3):
    src = _pad_idx(edge_index[0], 0)
    dst = _pad_idx(edge_index[1], TRASH)
    xa = x[:, :DH]
    xb = x[:, DH:]
    h0 = _sc_agg(xa, xb, src, dst)[:N]
    row = lambda v: v.reshape(1, -1)
    return pl.pallas_call(
        _mlp_kernel,
        out_shape=jax.ShapeDtypeStruct((N, D), jnp.float32),
    )(h0, W1.T, row(b1), row(g1), row(be1),
      W2.T, row(b2), row(g2), row(be2), row(g3), row(be3))


# speedup vs baseline: 1.2077x; 1.1783x over previous
import functools

import jax
import jax.numpy as jnp
from jax import lax
from jax.experimental import pallas as pl
from jax.experimental.pallas import tpu as pltpu
from jax.experimental.pallas import tpu_sc as plsc

N = 10000
E = 320000
D = 128
DH = D // 2
BN_EPS = 1e-5

NC = 2
NS = 16
EPT = E // NS
CHUNK = 80
NCHUNK = (EPT + CHUNK - 1) // CHUNK
EPTP = NCHUNK * CHUNK
TRASH = N
NPAD = 10240
RPT = NPAD // NS
XPT = N // NS
PDEPTH = 5
NBUF = 9


def _sc_agg_kernel(xa_hbm, xb_hbm, src_hbm, dst_hbm, out_hbm,
                   sidx, didx, rows, agg_sh, gsem, ssem):
    c = lax.axis_index("c")
    s = lax.axis_index("s")

    @pl.when(c == 0)
    def _():
        pltpu.sync_copy(xa_hbm.at[pl.ds(s * XPT, XPT)],
                        agg_sh.at[pl.ds(s * XPT, XPT)])

    @pl.when(c == 1)
    def _():
        pltpu.sync_copy(xb_hbm.at[pl.ds(s * XPT, XPT)],
                        agg_sh.at[pl.ds(s * XPT, XPT)])

    pltpu.sync_copy(src_hbm.at[s], sidx)
    pltpu.sync_copy(dst_hbm.at[s], didx)
    plsc.subcore_barrier()

    def make_loop(x_hbm):
        def body(i, _):
            @pl.when(i < NCHUNK)
            def _():
                k = lax.rem(i, NBUF)

                @pl.when(i >= NBUF)
                def _():
                    pltpu.make_async_copy(
                        rows.at[k], agg_sh.at[didx.at[0]], ssem.at[k]
                    ).wait()

                pltpu.make_async_copy(
                    x_hbm.at[sidx.at[i]], rows.at[k], gsem.at[k]
                ).start()

            j = i - PDEPTH

            @pl.when(j >= 0)
            def _():
                kj = lax.rem(j, NBUF)
                pltpu.make_async_copy(
                    x_hbm.at[sidx.at[j]], rows.at[kj], gsem.at[kj]
                ).wait()
                pltpu.make_async_copy(
                    rows.at[kj], agg_sh.at[didx.at[j]], ssem.at[kj]
                ).start(add=True)

            return 0

        lax.fori_loop(0, NCHUNK + PDEPTH, body, 0)

        def drain(k, _):
            pltpu.make_async_copy(
                rows.at[k], agg_sh.at[didx.at[0]], ssem.at[k]
            ).wait()
            return 0

        lax.fori_loop(0, NBUF, drain, 0)

    @pl.when(c == 0)
    def _():
        make_loop(xa_hbm)

    @pl.when(c == 1)
    def _():
        make_loop(xb_hbm)

    plsc.subcore_barrier()
    pl.delay(8000)
    plsc.subcore_barrier()

    pltpu.sync_copy(agg_sh.at[pl.ds(s * RPT, RPT)],
                    out_hbm.at[pl.ds(s * RPT, RPT), pl.ds(c * DH, DH)])


_sc_agg = functools.partial(
    pl.kernel,
    out_type=jax.ShapeDtypeStruct((NPAD, D), jnp.float32),
    mesh=plsc.VectorSubcoreMesh(core_axis_name="c", subcore_axis_name="s"),
    compiler_params=pltpu.CompilerParams(use_tc_tiling_on_sc=False),
    scratch_types=[
        pltpu.VMEM((NCHUNK, CHUNK), jnp.int32),
        pltpu.VMEM((NCHUNK, CHUNK), jnp.int32),
        pltpu.VMEM((NBUF, CHUNK, DH), jnp.float32),
        pltpu.VMEM_SHARED((NPAD, DH), jnp.float32),
        pltpu.SemaphoreType.DMA((NBUF,)),
        pltpu.SemaphoreType.DMA((NBUF,)),
    ],
)(_sc_agg_kernel)


def _mlp_kernel(h0_ref, w1t_ref, b1_ref, g1_ref, be1_ref,
                w2t_ref, b2_ref, g2_ref, be2_ref, g3_ref, be3_ref, out_ref):
    def bn(h, gamma, beta):
        mean = jnp.mean(h, axis=0, keepdims=True)
        var = jnp.mean((h - mean) ** 2, axis=0, keepdims=True)
        return (h - mean) * lax.rsqrt(var + BN_EPS) * gamma + beta

    h = jnp.dot(h0_ref[...], w1t_ref[...],
                preferred_element_type=jnp.float32) + b1_ref[...]
    h = jnp.maximum(bn(h, g1_ref[...], be1_ref[...]), 0.0)
    h = jnp.dot(h, w2t_ref[...],
                preferred_element_type=jnp.float32) + b2_ref[...]
    h = jnp.maximum(bn(h, g2_ref[...], be2_ref[...]), 0.0)
    out_ref[...] = bn(h, g3_ref[...], be3_ref[...])


def _pad_idx(a, fill):
    pad = jnp.full((NS, EPTP - EPT), fill, jnp.int32)
    return jnp.concatenate([a.reshape(NS, EPT), pad], axis=1) \
        .reshape(NS, NCHUNK, CHUNK)


def kernel(x, edge_index, W1, b1, g1, be1, W2, b2, g2, be2, g3, be3):
    src = _pad_idx(edge_index[0], 0)
    dst = _pad_idx(edge_index[1], TRASH)
    xa = x[:, :DH]
    xb = x[:, DH:]
    h0 = _sc_agg(xa, xb, src, dst)[:N]
    row = lambda v: v.reshape(1, -1)
    return pl.pallas_call(
        _mlp_kernel,
        out_shape=jax.ShapeDtypeStruct((N, D), jnp.float32),
    )(h0, W1.T, row(b1), row(g1), row(be1),
      W2.T, row(b2), row(g2), row(be2), row(g3), row(be3))


# speedup vs baseline: 1.2410x; 1.0276x over previous
import functools

import jax
import jax.numpy as jnp
from jax import lax
from jax.experimental import pallas as pl
from jax.experimental.pallas import tpu as pltpu
from jax.experimental.pallas import tpu_sc as plsc

N = 10000
E = 320000
D = 128
DH = D // 2
BN_EPS = 1e-5

NC = 2
NS = 16
EPT = E // NS
CHUNK = 80
NCHUNK = (EPT + CHUNK - 1) // CHUNK
EPTP = NCHUNK * CHUNK
TRASH = N
NPAD = 10240
RPT = NPAD // NS
XPT = N // NS
PDEPTH = 5
NBUF = 9


def _sc_agg_kernel(xa_hbm, xb_hbm, src_hbm, dst_hbm, out_hbm,
                   sidx, didx, rows, agg_sh, gsem, ssem):
    c = lax.axis_index("c")
    s = lax.axis_index("s")

    @pl.when(c == 0)
    def _():
        pltpu.sync_copy(xa_hbm.at[pl.ds(s * XPT, XPT)],
                        agg_sh.at[pl.ds(s * XPT, XPT)])

    @pl.when(c == 1)
    def _():
        pltpu.sync_copy(xb_hbm.at[pl.ds(s * XPT, XPT)],
                        agg_sh.at[pl.ds(s * XPT, XPT)])

    pltpu.sync_copy(src_hbm.at[s], sidx)
    pltpu.sync_copy(dst_hbm.at[s], didx)
    plsc.subcore_barrier()

    def make_loop(xcols):
        def body(i, _):
            @pl.when(i < NCHUNK)
            def _():
                k = lax.rem(i, NBUF)

                @pl.when(i >= NBUF)
                def _():
                    pltpu.make_async_copy(
                        rows.at[k], agg_sh.at[didx.at[0]], ssem.at[k]
                    ).wait()

                pltpu.make_async_copy(
                    xcols.at[sidx.at[i]], rows.at[k], gsem.at[k]
                ).start()

            j = i - PDEPTH

            @pl.when(j >= 0)
            def _():
                kj = lax.rem(j, NBUF)
                pltpu.make_async_copy(
                    xcols.at[sidx.at[j]], rows.at[kj], gsem.at[kj]
                ).wait()
                pltpu.make_async_copy(
                    rows.at[kj], agg_sh.at[didx.at[j]], ssem.at[kj]
                ).start(add=True)

            return 0

        lax.fori_loop(0, NCHUNK + PDEPTH, body, 0)

        def drain(k, _):
            pltpu.make_async_copy(
                rows.at[k], agg_sh.at[didx.at[0]], ssem.at[k]
            ).wait()
            return 0

        lax.fori_loop(0, NBUF, drain, 0)

    @pl.when(c == 0)
    def _():
        make_loop(xa_hbm)

    @pl.when(c == 1)
    def _():
        make_loop(xb_hbm)

    plsc.subcore_barrier()
    pl.delay(8000)
    plsc.subcore_barrier()

    pltpu.sync_copy(agg_sh.at[pl.ds(s * RPT, RPT)],
                    out_hbm.at[pl.ds(s * RPT, RPT), pl.ds(c * DH, DH)])


_sc_agg = functools.partial(
    pl.kernel,
    out_type=jax.ShapeDtypeStruct((NPAD, D), jnp.float32),
    mesh=plsc.VectorSubcoreMesh(core_axis_name="c", subcore_axis_name="s"),
    compiler_params=pltpu.CompilerParams(use_tc_tiling_on_sc=False),
    scratch_types=[
        pltpu.VMEM((NCHUNK, CHUNK), jnp.int32),
        pltpu.VMEM((NCHUNK, CHUNK), jnp.int32),
        pltpu.VMEM((NBUF, CHUNK, DH), jnp.float32),
        pltpu.VMEM_SHARED((NPAD, DH), jnp.float32),
        pltpu.SemaphoreType.DMA((NBUF,)),
        pltpu.SemaphoreType.DMA((NBUF,)),
    ],
)(_sc_agg_kernel)


def _mlp_kernel(h0_ref, w1t_ref, b1_ref, g1_ref, be1_ref,
                w2t_ref, b2_ref, g2_ref, be2_ref, g3_ref, be3_ref, out_ref):
    def bn(h, gamma, beta):
        mean = jnp.mean(h, axis=0, keepdims=True)
        var = jnp.mean((h - mean) ** 2, axis=0, keepdims=True)
        return (h - mean) * lax.rsqrt(var + BN_EPS) * gamma + beta

    h = jnp.dot(h0_ref[pl.ds(0, N), :], w1t_ref[...],
                preferred_element_type=jnp.float32) + b1_ref[...]
    h = jnp.maximum(bn(h, g1_ref[...], be1_ref[...]), 0.0)
    h = jnp.dot(h, w2t_ref[...],
                preferred_element_type=jnp.float32) + b2_ref[...]
    h = jnp.maximum(bn(h, g2_ref[...], be2_ref[...]), 0.0)
    out_ref[...] = bn(h, g3_ref[...], be3_ref[...])


def _pad_idx(a, fill):
    pad = jnp.full((NS, EPTP - EPT), fill, jnp.int32)
    return jnp.concatenate([a.reshape(NS, EPT), pad], axis=1) \
        .reshape(NS, NCHUNK, CHUNK)


def kernel(x, edge_index, W1, b1, g1, be1, W2, b2, g2, be2, g3, be3):
    src = _pad_idx(edge_index[0], 0)
    dst = _pad_idx(edge_index[1], TRASH)
    xa = x[:, :DH]
    xb = x[:, DH:]
    h0 = _sc_agg(xa, xb, src, dst)
    row = lambda v: v.reshape(1, -1)
    return pl.pallas_call(
        _mlp_kernel,
        out_shape=jax.ShapeDtypeStruct((N, D), jnp.float32),
    )(h0, W1.T, row(b1), row(g1), row(be1),
      W2.T, row(b2), row(g2), row(be2), row(g3), row(be3))
